# baseline (speedup 1.0000x reference)
import jax
import jax.numpy as jnp
from jax.experimental import pallas as pl
from jax.experimental.pallas import tpu as pltpu

_BLOCK = 10000


def _combine_kernel(wlt_ref, blt_ref, w1_ref, b1_ref, wc_ref, bc_ref):
    wc_ref[...] = jnp.dot(wlt_ref[...], w1_ref[...], preferred_element_type=jnp.float32)
    bc_ref[...] = jnp.dot(blt_ref[...], w1_ref[...], preferred_element_type=jnp.float32) + b1_ref[...]


def _mlp_kernel(f_ref, wc_ref, bc_ref, w2_ref, b2_ref, o_ref):
    f = f_ref[...].astype(jnp.bfloat16)
    h = jnp.dot(f, wc_ref[...].astype(jnp.bfloat16),
                preferred_element_type=jnp.float32) + bc_ref[...]
    h = jnp.maximum(h, 0.0).astype(jnp.bfloat16)
    o_ref[...] = jnp.dot(h, w2_ref[...].astype(jnp.bfloat16),
                         preferred_element_type=jnp.float32) + b2_ref[...]


def kernel(features, points, nuv, ranges, W_lt, b_lt, W1, b1, W2, b2):
    del points, nuv, ranges
    n, d_in = features.shape
    d_out = W_lt.shape[0]
    wlt = W_lt.T
    w1 = W1.T
    w2 = W2.T
    blt = b_lt[None, :]
    b1r = b1[None, :]
    b2r = b2[None, :]

    wc, bc = pl.pallas_call(
        _combine_kernel,
        out_shape=(
            jax.ShapeDtypeStruct((d_in, d_out), jnp.float32),
            jax.ShapeDtypeStruct((1, d_out), jnp.float32),
        ),
    )(wlt, blt, w1, b1r)

    weight_spec = lambda shape: pl.BlockSpec(shape, lambda i: (0, 0))
    return pl.pallas_call(
        _mlp_kernel,
        grid=(pl.cdiv(n, _BLOCK),),
        in_specs=[
            pl.BlockSpec((_BLOCK, d_in), lambda i: (i, 0)),
            weight_spec((d_in, d_out)),
            weight_spec((1, d_out)),
            weight_spec((d_out, d_out)),
            weight_spec((1, d_out)),
        ],
        out_specs=pl.BlockSpec((_BLOCK, d_out), lambda i: (i, 0)),
        out_shape=jax.ShapeDtypeStruct((n, d_out), jnp.float32),
        compiler_params=pltpu.CompilerParams(
            dimension_semantics=("parallel",),
        ),
    )(features, wc, bc, w2, b2r)


# speedup vs baseline: 1.1337x; 1.1337x over previous
import jax
import jax.numpy as jnp
from jax.experimental import pallas as pl
from jax.experimental.pallas import tpu as pltpu

_BLOCK = 20000


def _combine_kernel(wlt_ref, blt_ref, w1_ref, b1_ref, wc_ref, bc_ref):
    wc_ref[...] = jnp.dot(wlt_ref[...], w1_ref[...], preferred_element_type=jnp.float32)
    bc_ref[...] = jnp.dot(blt_ref[...], w1_ref[...], preferred_element_type=jnp.float32) + b1_ref[...]


def _mlp_kernel(f_ref, wc_ref, bc_ref, w2_ref, b2_ref, o_ref):
    f = f_ref[...]
    h = jnp.dot(f, wc_ref[...], preferred_element_type=jnp.float32) + bc_ref[...]
    h = jnp.maximum(h, 0.0)
    o_ref[...] = jnp.dot(h, w2_ref[...], preferred_element_type=jnp.float32) + b2_ref[...]


def kernel(features, points, nuv, ranges, W_lt, b_lt, W1, b1, W2, b2):
    del points, nuv, ranges
    n, d_in = features.shape
    d_out = W_lt.shape[0]
    wlt = W_lt.T
    w1 = W1.T
    w2 = W2.T
    blt = b_lt[None, :]
    b1r = b1[None, :]
    b2r = b2[None, :]

    wc, bc = pl.pallas_call(
        _combine_kernel,
        out_shape=(
            jax.ShapeDtypeStruct((d_in, d_out), jnp.float32),
            jax.ShapeDtypeStruct((1, d_out), jnp.float32),
        ),
    )(wlt, blt, w1, b1r)

    weight_spec = lambda shape: pl.BlockSpec(shape, lambda i: (0, 0))
    return pl.pallas_call(
        _mlp_kernel,
        grid=(pl.cdiv(n, _BLOCK),),
        in_specs=[
            pl.BlockSpec((_BLOCK, d_in), lambda i: (i, 0)),
            weight_spec((d_in, d_out)),
            weight_spec((1, d_out)),
            weight_spec((d_out, d_out)),
            weight_spec((1, d_out)),
        ],
        out_specs=pl.BlockSpec((_BLOCK, d_out), lambda i: (i, 0)),
        out_shape=jax.ShapeDtypeStruct((n, d_out), jnp.float32),
        compiler_params=pltpu.CompilerParams(
            dimension_semantics=("parallel",),
        ),
    )(features, wc, bc, w2, b2r)
